# baseline (speedup 1.0000x reference)
import functools

import jax
import jax.numpy as jnp
from jax import lax
from jax.experimental import pallas as pl
from jax.experimental.pallas import tpu as pltpu
from jax.experimental.pallas import tpu_sc as plsc

BATCH = 16384
FACTOR = 32

_info = plsc.get_sparse_core_info()
_NC, _NS, _L = _info.num_cores, _info.num_subcores, _info.num_lanes
_NW = _NC * _NS
_BPW = BATCH // _NW
_CHUNK = 128
_NCHUNK = _BPW // _CHUNK

_mesh = plsc.VectorSubcoreMesh(core_axis_name="c", subcore_axis_name="s")


@functools.partial(
    pl.kernel,
    mesh=_mesh,
    compiler_params=pltpu.CompilerParams(
        use_tc_tiling_on_sc=False, needs_layout_passes=False),
    out_type=(
        jax.ShapeDtypeStruct((BATCH,), jnp.float32),
        jax.ShapeDtypeStruct((BATCH,), jnp.float32),
    ),
    scratch_types=[
        pltpu.VMEM((_NCHUNK, _CHUNK), jnp.int32),
        pltpu.VMEM((_NCHUNK, _CHUNK), jnp.int32),
        pltpu.VMEM((_NCHUNK, _CHUNK), jnp.int32),
        pltpu.VMEM((_BPW, FACTOR), jnp.float32),
        pltpu.VMEM((_BPW, FACTOR), jnp.float32),
        pltpu.VMEM((_BPW, FACTOR), jnp.float32),
        pltpu.VMEM((_BPW,), jnp.float32),
        pltpu.VMEM((_BPW,), jnp.float32),
        pltpu.SemaphoreType.DMA,
        pltpu.SemaphoreType.DMA,
    ],
)
def _bpr_sc(user_hbm, item_i_hbm, item_j_hbm, uw_hbm, iw_hbm,
            out_i_hbm, out_j_hbm,
            u_idx, i_idx, j_idx, urows, irows, jrows,
            pred_i_v, pred_j_v, sem_idx, sem_rows):
    wid = lax.axis_index("s") * _NC + lax.axis_index("c")
    base = wid * _BPW

    idx_copies = []
    for k in range(_NCHUNK):
        off = base + k * _CHUNK
        idx_copies.append(
            pltpu.async_copy(user_hbm.at[pl.ds(off, _CHUNK)], u_idx.at[k],
                             sem_idx))
        idx_copies.append(
            pltpu.async_copy(item_i_hbm.at[pl.ds(off, _CHUNK)], i_idx.at[k],
                             sem_idx))
        idx_copies.append(
            pltpu.async_copy(item_j_hbm.at[pl.ds(off, _CHUNK)], j_idx.at[k],
                             sem_idx))
    for cp in idx_copies:
        cp.wait()

    row_copies = []
    for k in range(_NCHUNK):
        dst = pl.ds(k * _CHUNK, _CHUNK)
        row_copies.append(
            pltpu.async_copy(uw_hbm.at[u_idx.at[k]], urows.at[dst], sem_rows))
        row_copies.append(
            pltpu.async_copy(iw_hbm.at[i_idx.at[k]], irows.at[dst], sem_rows))
        row_copies.append(
            pltpu.async_copy(iw_hbm.at[j_idx.at[k]], jrows.at[dst], sem_rows))
    for cp in row_copies:
        cp.wait()

    lane = lax.iota(jnp.int32, 16)

    def group_body(g, carry):
        row0 = g * 16
        row_idx = lane + row0
        acc_i = jnp.zeros((16,), jnp.float32)
        acc_j = jnp.zeros((16,), jnp.float32)
        for c in range(FACTOR):
            col = jnp.bitwise_and(lane + c, FACTOR - 1)
            u = plsc.load_gather(urows, [row_idx, col])
            ei = plsc.load_gather(irows, [row_idx, col])
            ej = plsc.load_gather(jrows, [row_idx, col])
            acc_i = acc_i + u * ei
            acc_j = acc_j + u * ej
        pred_i_v[pl.ds(row0, 16)] = acc_i
        pred_j_v[pl.ds(row0, 16)] = acc_j
        return carry

    lax.fori_loop(0, _BPW // 16, group_body, 0)

    pltpu.sync_copy(pred_i_v, out_i_hbm.at[pl.ds(base, _BPW)])
    pltpu.sync_copy(pred_j_v, out_j_hbm.at[pl.ds(base, _BPW)])


def kernel(user, item_i, item_j, embed_user_weight, embed_item_weight):
    user = user.astype(jnp.int32)
    item_i = item_i.astype(jnp.int32)
    item_j = item_j.astype(jnp.int32)
    return _bpr_sc(user, item_i, item_j, embed_user_weight, embed_item_weight)


# speedup vs baseline: 1.4586x; 1.4586x over previous
import functools

import jax
import jax.numpy as jnp
from jax import lax
from jax.experimental import pallas as pl
from jax.experimental.pallas import tpu as pltpu
from jax.experimental.pallas import tpu_sc as plsc

BATCH = 16384
FACTOR = 32

_info = plsc.get_sparse_core_info()
_NC, _NS, _L = _info.num_cores, _info.num_subcores, _info.num_lanes
_NW = _NC * _NS
_BPW = BATCH // _NW
_CHUNK = 128
_NCHUNK = _BPW // _CHUNK

_mesh = plsc.VectorSubcoreMesh(core_axis_name="c", subcore_axis_name="s")


@functools.partial(
    pl.kernel,
    mesh=_mesh,
    compiler_params=pltpu.CompilerParams(needs_layout_passes=False),
    out_type=(
        jax.ShapeDtypeStruct((BATCH,), jnp.float32),
        jax.ShapeDtypeStruct((BATCH,), jnp.float32),
    ),
    scratch_types=[
        pltpu.VMEM((_BPW,), jnp.int32),
        pltpu.VMEM((_BPW,), jnp.int32),
        pltpu.VMEM((_BPW,), jnp.int32),
        pltpu.VMEM((_CHUNK, FACTOR), jnp.float32),
        pltpu.VMEM((_CHUNK, FACTOR), jnp.float32),
        pltpu.VMEM((_CHUNK, FACTOR), jnp.float32),
        pltpu.VMEM((_BPW,), jnp.float32),
        pltpu.VMEM((_BPW,), jnp.float32),
        pltpu.SemaphoreType.DMA,
        pltpu.SemaphoreType.DMA,
    ],
)
def _bpr_sc(user_hbm, item_i_hbm, item_j_hbm, uw_hbm, iw_hbm,
            out_i_hbm, out_j_hbm,
            u_idx, i_idx, j_idx, urows, irows, jrows,
            pred_i_v, pred_j_v, sem_idx, sem_rows):
    wid = lax.axis_index("s") * _NC + lax.axis_index("c")
    base = wid * _BPW

    cp_u = pltpu.async_copy(user_hbm.at[pl.ds(base, _BPW)], u_idx, sem_idx)
    cp_i = pltpu.async_copy(item_i_hbm.at[pl.ds(base, _BPW)], i_idx, sem_idx)
    cp_j = pltpu.async_copy(item_j_hbm.at[pl.ds(base, _BPW)], j_idx, sem_idx)
    cp_u.wait()
    cp_i.wait()
    cp_j.wait()

    lane = lax.iota(jnp.int32, 16)

    def chunk_body(ch, carry):
        c0 = ch * _CHUNK

        copies = []
        for w in range(_CHUNK // 16):
            r0 = c0 + w * 16
            uvec = u_idx[pl.ds(r0, 16)]
            ivec = i_idx[pl.ds(r0, 16)]
            jvec = j_idx[pl.ds(r0, 16)]
            for t in range(16):
                r = w * 16 + t
                copies.append(pltpu.async_copy(
                    uw_hbm.at[pl.ds(uvec[t], 1), :],
                    urows.at[pl.ds(r, 1), :], sem_rows))
                copies.append(pltpu.async_copy(
                    iw_hbm.at[pl.ds(ivec[t], 1), :],
                    irows.at[pl.ds(r, 1), :], sem_rows))
                copies.append(pltpu.async_copy(
                    iw_hbm.at[pl.ds(jvec[t], 1), :],
                    jrows.at[pl.ds(r, 1), :], sem_rows))
        for cp in copies:
            cp.wait()

        def group_body(g, inner):
            row0 = g * 16
            row_idx = lane + row0
            acc_i = jnp.zeros((16,), jnp.float32)
            acc_j = jnp.zeros((16,), jnp.float32)
            for c in range(FACTOR):
                col = jnp.bitwise_and(lane + c, FACTOR - 1)
                u = plsc.load_gather(urows, [row_idx, col])
                ei = plsc.load_gather(irows, [row_idx, col])
                ej = plsc.load_gather(jrows, [row_idx, col])
                acc_i = acc_i + u * ei
                acc_j = acc_j + u * ej
            pred_i_v[pl.ds(c0 + row0, 16)] = acc_i
            pred_j_v[pl.ds(c0 + row0, 16)] = acc_j
            return inner

        lax.fori_loop(0, _CHUNK // 16, group_body, 0)
        return carry

    lax.fori_loop(0, _NCHUNK, chunk_body, 0)

    pltpu.sync_copy(pred_i_v, out_i_hbm.at[pl.ds(base, _BPW)])
    pltpu.sync_copy(pred_j_v, out_j_hbm.at[pl.ds(base, _BPW)])


def kernel(user, item_i, item_j, embed_user_weight, embed_item_weight):
    user = user.astype(jnp.int32)
    item_i = item_i.astype(jnp.int32)
    item_j = item_j.astype(jnp.int32)
    return _bpr_sc(user, item_i, item_j, embed_user_weight, embed_item_weight)
